# baseline (speedup 1.0000x reference)
import functools

import jax
import jax.numpy as jnp
from jax import lax
from jax.experimental import pallas as pl
from jax.experimental.pallas import tpu as pltpu
from jax.experimental.pallas import tpu_sc as plsc

N = 16384
K = 8192
D = 32
COMMIT_W = 0.25
BN = 128
NB = N // BN
HK = K // 2


def _half_argmin(d, base):
    v = d[:, base:base + 128]
    j = jnp.zeros((BN, 128), jnp.int32)
    for g in range(1, HK // 128):
        c = d[:, base + g * 128:base + (g + 1) * 128]
        lt = c < v
        j = jnp.where(lt, g, j)
        v = jnp.minimum(c, v)
    col = j * 128 + lax.broadcasted_iota(jnp.int32, (BN, 128), 1)
    md = jnp.min(v, axis=1, keepdims=True)
    ih = jnp.min(jnp.where(v == md, col, K), axis=1)
    return md[:, 0], ih


def _argmin_body(x_ref, wt_ref, x2_ref, w2_ref, idx_ref, loss_ref, acc_ref):
    i = pl.program_id(0)
    x_blk = x_ref[...]
    wt = wt_ref[...]
    m = jax.lax.dot_general(
        x_blk, wt, (((1,), (0,)), ((), ())),
        preferred_element_type=jnp.float32,
    )
    d2 = (x2_ref[...] + w2_ref[...]) - 2.0 * m
    d = jnp.sqrt(jnp.maximum(d2, 0.0))
    m0, i0 = _half_argmin(d, 0)
    m1, i1 = _half_argmin(d, HK)
    a0 = m0.astype(jnp.bfloat16).astype(jnp.float32)
    win1 = m1 < a0
    idx = jnp.where(win1, i1 + HK, i0)
    idx_ref[0, 0, :] = idx
    msel = jnp.where(win1, m1, m0)
    part = jnp.sum(msel * msel)

    @pl.when(i == 0)
    def _():
        acc_ref[0] = 0.0

    acc_ref[0] += part

    @pl.when(i == NB - 1)
    def _():
        loss_ref[...] = jnp.full((1, 1), acc_ref[0], jnp.float32)


def _distance_argmin(x, weight):
    x2 = jnp.sum(x * x, axis=-1, keepdims=True)
    w2 = jnp.sum(weight * weight, axis=-1)[None, :]
    wt = weight.T
    idx3, loss_sum = pl.pallas_call(
        _argmin_body,
        grid=(NB,),
        in_specs=[
            pl.BlockSpec((BN, D), lambda i: (i, 0)),
            pl.BlockSpec((D, K), lambda i: (0, 0)),
            pl.BlockSpec((BN, 1), lambda i: (i, 0)),
            pl.BlockSpec((1, K), lambda i: (0, 0)),
        ],
        out_specs=[
            pl.BlockSpec((1, 1, BN), lambda i: (i, 0, 0)),
            pl.BlockSpec((1, 1), lambda i: (0, 0)),
        ],
        out_shape=[
            jax.ShapeDtypeStruct((NB, 1, BN), jnp.int32),
            jax.ShapeDtypeStruct((1, 1), jnp.float32),
        ],
        scratch_shapes=[pltpu.SMEM((1,), jnp.float32)],
    )(x, wt, x2, w2)
    return idx3.reshape(N), loss_sum[0, 0]


GD = 128


def _sc_gather(weight, idx):
    info = plsc.get_sparse_core_info()
    nw = info.num_cores * info.num_subcores
    bpw = N // nw
    mesh = plsc.VectorSubcoreMesh(core_axis_name="c", subcore_axis_name="s")
    wp = jnp.pad(weight, ((0, 0), (0, GD - D)))

    @functools.partial(
        pl.kernel,
        mesh=mesh,
        out_type=jax.ShapeDtypeStruct((N, GD), jnp.float32),
        scratch_types=[
            pltpu.VMEM((bpw,), jnp.int32),
            pltpu.VMEM((bpw, GD), jnp.float32),
            pltpu.SemaphoreType.DMA,
        ],
    )
    def gather_kernel(table_hbm, idx_hbm, out_hbm, idx_v, rows_v, sem):
        wid = lax.axis_index("s") * info.num_cores + lax.axis_index("c")
        base = wid * bpw
        pltpu.sync_copy(idx_hbm.at[pl.ds(base, bpw)], idx_v)
        pltpu.async_copy(table_hbm.at[idx_v], rows_v, sem).wait()
        pltpu.sync_copy(rows_v, out_hbm.at[pl.ds(base, bpw)])

    return gather_kernel(wp, idx)[:, :D]


def kernel(x, weight):
    idx, loss_sum = _distance_argmin(x, weight)
    quantised = _sc_gather(weight, idx)
    loss = (1.0 + COMMIT_W) * loss_sum / (N * D)
    quantised_st = x + lax.stop_gradient(quantised - x)
    return (quantised_st, loss, idx)


# speedup vs baseline: 1.1179x; 1.1179x over previous
import functools

import jax
import jax.numpy as jnp
from jax import lax
from jax.experimental import pallas as pl
from jax.experimental.pallas import tpu as pltpu
from jax.experimental.pallas import tpu_sc as plsc

N = 16384
K = 8192
D = 32
COMMIT_W = 0.25
BN = 512
NB = N // BN
HK = K // 2


def _half_argmin(d, base):
    v = d[:, base:base + 128]
    j = jnp.zeros((BN, 128), jnp.int32)
    for g in range(1, HK // 128):
        c = d[:, base + g * 128:base + (g + 1) * 128]
        lt = c < v
        j = jnp.where(lt, g, j)
        v = jnp.minimum(c, v)
    col = j * 128 + lax.broadcasted_iota(jnp.int32, (BN, 128), 1)
    md = jnp.min(v, axis=1, keepdims=True)
    ih = jnp.min(jnp.where(v == md, col, K), axis=1)
    return md[:, 0], ih


def _argmin_body(x_ref, wt_ref, x2_ref, w2_ref, idx_ref, loss_ref, acc_ref):
    i = pl.program_id(0)
    x_blk = x_ref[...]
    wt = wt_ref[...]
    m = jax.lax.dot_general(
        x_blk, wt, (((1,), (0,)), ((), ())),
        preferred_element_type=jnp.float32,
    )
    d2 = (x2_ref[...] + w2_ref[...]) - 2.0 * m
    d = jnp.sqrt(jnp.maximum(d2, 0.0))
    m0, i0 = _half_argmin(d, 0)
    m1, i1 = _half_argmin(d, HK)
    a0 = m0.astype(jnp.bfloat16).astype(jnp.float32)
    win1 = m1 < a0
    idx = jnp.where(win1, i1 + HK, i0)
    idx_ref[0, 0, :] = idx
    msel = jnp.where(win1, m1, m0)
    part = jnp.sum(msel * msel)

    @pl.when(i == 0)
    def _():
        acc_ref[0] = 0.0

    acc_ref[0] += part

    @pl.when(i == NB - 1)
    def _():
        loss_ref[...] = jnp.full((1, 1), acc_ref[0], jnp.float32)


def _distance_argmin(x, weight):
    x2 = jnp.sum(x * x, axis=-1, keepdims=True)
    w2 = jnp.sum(weight * weight, axis=-1)[None, :]
    wt = weight.T
    idx3, loss_sum = pl.pallas_call(
        _argmin_body,
        grid=(NB,),
        in_specs=[
            pl.BlockSpec((BN, D), lambda i: (i, 0)),
            pl.BlockSpec((D, K), lambda i: (0, 0)),
            pl.BlockSpec((BN, 1), lambda i: (i, 0)),
            pl.BlockSpec((1, K), lambda i: (0, 0)),
        ],
        out_specs=[
            pl.BlockSpec((1, 1, BN), lambda i: (i, 0, 0)),
            pl.BlockSpec((1, 1), lambda i: (0, 0)),
        ],
        out_shape=[
            jax.ShapeDtypeStruct((NB, 1, BN), jnp.int32),
            jax.ShapeDtypeStruct((1, 1), jnp.float32),
        ],
        scratch_shapes=[pltpu.SMEM((1,), jnp.float32)],
    )(x, wt, x2, w2)
    return idx3.reshape(N), loss_sum[0, 0]


GD = 128


def _sc_gather(weight, idx):
    info = plsc.get_sparse_core_info()
    nw = info.num_cores * info.num_subcores
    bpw = N // nw
    mesh = plsc.VectorSubcoreMesh(core_axis_name="c", subcore_axis_name="s")
    wp = jnp.pad(weight, ((0, 0), (0, GD - D)))

    @functools.partial(
        pl.kernel,
        mesh=mesh,
        out_type=jax.ShapeDtypeStruct((N, GD), jnp.float32),
        scratch_types=[
            pltpu.VMEM((bpw,), jnp.int32),
            pltpu.VMEM((bpw, GD), jnp.float32),
            pltpu.SemaphoreType.DMA,
        ],
    )
    def gather_kernel(table_hbm, idx_hbm, out_hbm, idx_v, rows_v, sem):
        wid = lax.axis_index("s") * info.num_cores + lax.axis_index("c")
        base = wid * bpw
        pltpu.sync_copy(idx_hbm.at[pl.ds(base, bpw)], idx_v)
        pltpu.async_copy(table_hbm.at[idx_v], rows_v, sem).wait()
        pltpu.sync_copy(rows_v, out_hbm.at[pl.ds(base, bpw)])

    return gather_kernel(wp, idx)[:, :D]


def kernel(x, weight):
    idx, loss_sum = _distance_argmin(x, weight)
    quantised = _sc_gather(weight, idx)
    loss = (1.0 + COMMIT_W) * loss_sum / (N * D)
    quantised_st = x + lax.stop_gradient(quantised - x)
    return (quantised_st, loss, idx)


# speedup vs baseline: 1.1666x; 1.0436x over previous
import functools

import jax
import jax.numpy as jnp
from jax import lax
from jax.experimental import pallas as pl
from jax.experimental.pallas import tpu as pltpu
from jax.experimental.pallas import tpu_sc as plsc

N = 16384
K = 8192
D = 32
COMMIT_W = 0.25
BN = 512
NB = N // BN
HK = K // 2


def _half_argmin(d, base):
    v = d[:, base:base + 128]
    j = jnp.zeros((BN, 128), jnp.int32)
    for g in range(1, HK // 128):
        c = d[:, base + g * 128:base + (g + 1) * 128]
        lt = c < v
        j = jnp.where(lt, g, j)
        v = jnp.minimum(c, v)
    col = j * 128 + lax.broadcasted_iota(jnp.int32, (BN, 128), 1)
    md = jnp.min(v, axis=1, keepdims=True)
    ih = jnp.min(jnp.where(v == md, col, K), axis=1)
    return md[:, 0], ih


def _argmin_body(x_ref, wt_ref, x2_ref, w2_ref, idx_ref, loss_ref, acc_ref):
    i = pl.program_id(0)
    x_blk = x_ref[...]
    wt = wt_ref[...]
    m2 = jax.lax.dot_general(
        x_blk, wt, (((1,), (0,)), ((), ())),
        preferred_element_type=jnp.float32,
    )
    d2 = (x2_ref[...] + w2_ref[...]) - m2
    d = jnp.sqrt(jnp.maximum(d2, 0.0))
    m0, i0 = _half_argmin(d, 0)
    m1, i1 = _half_argmin(d, HK)
    a0 = m0.astype(jnp.bfloat16).astype(jnp.float32)
    win1 = m1 < a0
    idx = jnp.where(win1, i1 + HK, i0)
    idx_ref[0, 0, :] = idx
    msel = jnp.where(win1, m1, m0)
    part = jnp.sum(msel * msel)

    @pl.when(i == 0)
    def _():
        acc_ref[0] = 0.0

    acc_ref[0] += part

    @pl.when(i == NB - 1)
    def _():
        loss_ref[...] = jnp.full((1, 1), acc_ref[0], jnp.float32)


def _distance_argmin(x, weight):
    x2 = jnp.sum(x * x, axis=-1, keepdims=True)
    w2 = jnp.sum(weight * weight, axis=-1)[None, :]
    wt = (2.0 * weight).T
    idx3, loss_sum = pl.pallas_call(
        _argmin_body,
        grid=(NB,),
        in_specs=[
            pl.BlockSpec((BN, D), lambda i: (i, 0)),
            pl.BlockSpec((D, K), lambda i: (0, 0)),
            pl.BlockSpec((BN, 1), lambda i: (i, 0)),
            pl.BlockSpec((1, K), lambda i: (0, 0)),
        ],
        out_specs=[
            pl.BlockSpec((1, 1, BN), lambda i: (i, 0, 0)),
            pl.BlockSpec((1, 1), lambda i: (0, 0)),
        ],
        out_shape=[
            jax.ShapeDtypeStruct((NB, 1, BN), jnp.int32),
            jax.ShapeDtypeStruct((1, 1), jnp.float32),
        ],
        scratch_shapes=[pltpu.SMEM((1,), jnp.float32)],
    )(x, wt, x2, w2)
    return idx3.reshape(N), loss_sum[0, 0]


GD = 128


def _sc_gather(weight, idx):
    info = plsc.get_sparse_core_info()
    nw = info.num_cores * info.num_subcores
    bpw = N // nw
    mesh = plsc.VectorSubcoreMesh(core_axis_name="c", subcore_axis_name="s")
    wp = jnp.pad(weight, ((0, 0), (0, GD - D)))

    @functools.partial(
        pl.kernel,
        mesh=mesh,
        out_type=jax.ShapeDtypeStruct((N, GD), jnp.float32),
        scratch_types=[
            pltpu.VMEM((bpw,), jnp.int32),
            pltpu.VMEM((bpw, GD), jnp.float32),
            pltpu.SemaphoreType.DMA,
        ],
    )
    def gather_kernel(table_hbm, idx_hbm, out_hbm, idx_v, rows_v, sem):
        wid = lax.axis_index("s") * info.num_cores + lax.axis_index("c")
        base = wid * bpw
        pltpu.sync_copy(idx_hbm.at[pl.ds(base, bpw)], idx_v)
        pltpu.async_copy(table_hbm.at[idx_v], rows_v, sem).wait()
        pltpu.sync_copy(rows_v, out_hbm.at[pl.ds(base, bpw)])

    return gather_kernel(wp, idx)[:, :D]


def kernel(x, weight):
    idx, loss_sum = _distance_argmin(x, weight)
    quantised = _sc_gather(weight, idx)
    loss = (1.0 + COMMIT_W) * loss_sum / (N * D)
    quantised_st = x + lax.stop_gradient(quantised - x)
    return (quantised_st, loss, idx)
